# baseline (speedup 1.0000x reference)
import functools

import jax
import jax.numpy as jnp
from jax import lax
from jax.experimental import pallas as pl
from jax.experimental.pallas import tpu as pltpu
from jax.experimental.pallas import tpu_sc as plsc

B, N, T = 1024, 26, 20
M = B * N * T
D = 16
V = 100000
NC, NS = 2, 16
NW = NC * NS
CHUNK = M // NW
SUB = 1040
N_ITERS = CHUNK // SUB

_mesh = plsc.VectorSubcoreMesh(core_axis_name="c", subcore_axis_name="s")


@functools.partial(
    pl.kernel,
    mesh=_mesh,
    compiler_params=pltpu.CompilerParams(use_tc_tiling_on_sc=False),
    out_type=jax.ShapeDtypeStruct((M, 3 * D), jnp.float32),
    scratch_types=[
        [[pltpu.VMEM((SUB,), jnp.int32)] * 3] * 2,
        [[pltpu.VMEM((SUB, D), jnp.float32)] * 3] * 2,
        [pltpu.SemaphoreType.DMA] * 2,
        [pltpu.SemaphoreType.DMA] * 2,
        [pltpu.SemaphoreType.DMA] * 2,
    ],
)
def _embed(xt, w0, w1, w2, out, idx_v, rows_v, sem_i, sem_g, sem_w):
    wid = lax.axis_index("s") * NC + lax.axis_index("c")
    base = wid * CHUNK
    tables = (w0, w1, w2)

    idx_cps = {}
    gathers = {}
    writes = {}

    def fetch_idx(j, s):
        p0 = base + j * SUB
        for i in range(3):
            idx_cps[(j, i)] = pltpu.async_copy(
                xt.at[i, pl.ds(p0, SUB)], idx_v[s][i], sem_i[s]
            )

    def start_gathers(j, s):
        for i in range(3):
            idx_cps[(j, i)].wait()
        for i in range(3):
            gathers[(j, i)] = pltpu.async_copy(
                tables[i].at[idx_v[s][i]], rows_v[s][i], sem_g[s]
            )

    def write_out(j, s):
        p0 = base + j * SUB
        for i in range(3):
            gathers[(j, i)].wait()
        for i in range(3):
            writes[(j, i)] = pltpu.async_copy(
                rows_v[s][i], out.at[pl.ds(p0, SUB), pl.ds(i * D, D)], sem_w[s]
            )

    fetch_idx(0, 0)
    for j in range(N_ITERS):
        s = j % 2
        if j >= 2:
            for i in range(3):
                writes[(j - 2, i)].wait()
        start_gathers(j, s)
        if j >= 1:
            write_out(j - 1, 1 - s)
        if j + 1 < N_ITERS:
            fetch_idx(j + 1, 1 - s)
    write_out(N_ITERS - 1, (N_ITERS - 1) % 2)
    for j in (N_ITERS - 2, N_ITERS - 1):
        for i in range(3):
            writes[(j, i)].wait()


def kernel(x, W0, W1, W2):
    xt = x.reshape(M, 3).T
    out = _embed(xt, W0, W1, W2)
    eye = jnp.eye(3 * D, dtype=jnp.float32)
    out = jax.lax.dot(out, eye, precision=jax.lax.Precision.HIGHEST)
    return out.reshape(B, N, T, 3 * D)


# speedup vs baseline: 2.0017x; 2.0017x over previous
import functools

import jax
import jax.numpy as jnp
from jax import lax
from jax.experimental import pallas as pl
from jax.experimental.pallas import tpu as pltpu
from jax.experimental.pallas import tpu_sc as plsc

B, N, T = 1024, 26, 20
D = 16
NC, NS = 2, 16
NW = NC * NS
NU = N * T
J_FULL = NU // NW
NX = NU - J_FULL * NW

_mesh = plsc.VectorSubcoreMesh(core_axis_name="c", subcore_axis_name="s")


@functools.partial(
    pl.kernel,
    mesh=_mesh,
    compiler_params=pltpu.CompilerParams(use_tc_tiling_on_sc=False),
    out_type=jax.ShapeDtypeStruct((N, T, B, 3 * D), jnp.float32),
    scratch_types=[
        [[pltpu.VMEM((B,), jnp.int32)] * 3] * 2,
        [[pltpu.VMEM((B, D), jnp.float32)] * 3] * 2,
        [pltpu.SemaphoreType.DMA] * 2,
        [pltpu.SemaphoreType.DMA] * 2,
        [pltpu.SemaphoreType.DMA] * 2,
    ],
)
def _embed(xnib, w0, w1, w2, out, idx_v, rows_v, sem_i, sem_g, sem_w):
    wid = lax.axis_index("s") * NC + lax.axis_index("c")
    tables = (w0, w1, w2)
    extra = wid < NX

    def unit_nt(j):
        u = wid + NW * j
        n = u // T
        return n, u - n * T

    idx_cps = {}
    gathers = {}
    writes = {}

    def fetch_idx(j, s):
        n, t = unit_nt(j)
        for i in range(3):
            idx_cps[(j, i)] = pltpu.async_copy(
                xnib.at[n, i, t], idx_v[s][i], sem_i[s]
            )

    def start_gathers(j, s):
        for i in range(3):
            idx_cps[(j, i)].wait()
        for i in range(3):
            gathers[(j, i)] = pltpu.async_copy(
                tables[i].at[idx_v[s][i]], rows_v[s][i], sem_g[s]
            )

    def write_out(j, s):
        n, t = unit_nt(j)
        for i in range(3):
            gathers[(j, i)].wait()
        for i in range(3):
            writes[(j, i)] = pltpu.async_copy(
                rows_v[s][i], out.at[n, t, :, pl.ds(i * D, D)], sem_w[s]
            )

    fetch_idx(0, 0)
    for j in range(J_FULL):
        s = j % 2
        if j >= 2:
            for i in range(3):
                writes[(j - 2, i)].wait()
        start_gathers(j, s)
        if j >= 1:
            write_out(j - 1, 1 - s)
        if j + 1 < J_FULL:
            fetch_idx(j + 1, 1 - s)

    s_last = (J_FULL - 1) % 2
    for i in range(3):
        writes[(J_FULL - 2, i)].wait()

    write_out(J_FULL - 1, s_last)

    @pl.when(extra)
    def _():
        se = 1 - s_last
        n, t = unit_nt(J_FULL)
        cps = [
            pltpu.async_copy(xnib.at[n, i, t], idx_v[se][i], sem_i[se])
            for i in range(3)
        ]
        for c in cps:
            c.wait()
        gs = [
            pltpu.async_copy(tables[i].at[idx_v[se][i]], rows_v[se][i], sem_g[se])
            for i in range(3)
        ]
        for g in gs:
            g.wait()
        ws = [
            pltpu.async_copy(rows_v[se][i], out.at[n, t, :, pl.ds(i * D, D)], sem_w[se])
            for i in range(3)
        ]
        for w in ws:
            w.wait()

    for i in range(3):
        writes[(J_FULL - 1, i)].wait()


def kernel(x, W0, W1, W2):
    xnib = jnp.transpose(x, (1, 3, 2, 0))
    out = _embed(xnib, W0, W1, W2)
    return jnp.transpose(out, (2, 0, 1, 3))
